# Initial kernel scaffold; baseline (speedup 1.0000x reference)
import jax
import jax.numpy as jnp
from jax.experimental import pallas as pl


def kernel(images):
    raise NotImplementedError("write your pallas kernel here")


# speedup vs baseline: 17.9560x; 17.9560x over previous
import jax
import jax.numpy as jnp
from jax import lax
from jax.experimental import pallas as pl


def _body(x_ref, o_ref):
    n_h, n_w, p, pc = 24, 24, 16, 48
    N, D = n_h * n_w, p * pc
    xb = x_ref[0]
    patches = xb.reshape(n_h, p, n_w, pc).transpose(0, 2, 1, 3).reshape(N, D)
    mx = jnp.max(patches, axis=1, keepdims=True)
    mask = mx > 0.0
    maskf = mask.astype(jnp.float32)
    count = jnp.sum(maskf)

    def fast(_):
        return patches

    def slow(_):
        row = lax.broadcasted_iota(jnp.int32, (N, N), 0)
        col = lax.broadcasted_iota(jnp.int32, (N, N), 1).astype(jnp.float32)
        tri = (col < row.astype(jnp.float32)).astype(jnp.float32)
        psum_ex = jnp.dot(tri, maskf, preferred_element_type=jnp.float32)
        n_idx = lax.broadcasted_iota(jnp.int32, (N, 1), 0).astype(jnp.float32)
        dest = jnp.where(mask, psum_ex, count + (n_idx - psum_ex))
        onehot_t = (dest == col).astype(jnp.float32)
        masked = patches * maskf
        return lax.dot_general(
            onehot_t, masked, (((0,), (0,)), ((), ())),
            preferred_element_type=jnp.float32)

    o_ref[0] = lax.cond(count == float(N), fast, slow, None)


def kernel(images):
    B, H, W, C = images.shape
    p = 16
    n_h, n_w = H // p, W // p
    N, D = n_h * n_w, p * p * C
    x = images.reshape(B, H, W * C)
    out = pl.pallas_call(
        _body,
        grid=(B,),
        in_specs=[pl.BlockSpec((1, H, W * C), lambda b: (b, 0, 0))],
        out_specs=pl.BlockSpec((1, N, D), lambda b: (b, 0, 0)),
        out_shape=jax.ShapeDtypeStruct((B, N, D), jnp.float32),
    )(x)
    return out.reshape(B, N, p, p, C)
